# baseline (speedup 1.0000x reference)
import jax
import jax.numpy as jnp
from jax import lax
from jax.experimental import pallas as pl

_TI = 16


def _fused_kernel(adj_ref, uadj_ref, alpha_ref, uops_ref, adj_out_ref, ops_out_ref):
    ti = pl.program_id(1)
    tj = pl.program_id(2)
    nt = pl.num_programs(2)

    a = adj_ref[...]
    t = -jnp.log(uadj_ref[...])
    act = 1.0 / (1.0 + t * jnp.exp(-a))
    i = ti * _TI + lax.broadcasted_iota(jnp.int32, a.shape, 0)
    j = tj * _TI + lax.broadcasted_iota(jnp.int32, a.shape, 1)
    adj_out_ref[...] = jnp.where(j > i, act, 0.0)

    @pl.when(tj == nt - 1)
    def _ops():
        e = jnp.exp(alpha_ref[...]) / (-jnp.log(uops_ref[...]))
        ops_out_ref[...] = e / jnp.sum(e, axis=1, keepdims=True)


def kernel(adj_para, ops_alpha, u_adj, u_ops):
    B, N, _ = adj_para.shape
    OPS = ops_alpha.shape[-1]

    adj_t = jnp.transpose(adj_para, (1, 2, 0))
    uadj_t = jnp.transpose(u_adj, (1, 2, 0))
    alpha_t = jnp.transpose(ops_alpha, (1, 2, 0))
    uops_t = jnp.transpose(u_ops, (1, 2, 0))

    nt = N // _TI
    NB = 2
    BT = B // NB
    adj_in_spec = pl.BlockSpec((_TI, _TI, BT), lambda bk, ti, tj: (ti, jnp.maximum(tj, ti), bk))
    adj_out_spec = pl.BlockSpec((_TI, _TI, BT), lambda bk, ti, tj: (ti, tj, bk))
    ops_spec = pl.BlockSpec((_TI, OPS, BT), lambda bk, ti, tj: (ti, 0, bk))

    adj_out_t, ops_out_t = pl.pallas_call(
        _fused_kernel,
        grid=(NB, nt, nt),
        in_specs=[adj_in_spec, adj_in_spec, ops_spec, ops_spec],
        out_specs=[adj_out_spec, ops_spec],
        out_shape=[
            jax.ShapeDtypeStruct((N, N, B), adj_para.dtype),
            jax.ShapeDtypeStruct((N, OPS, B), ops_alpha.dtype),
        ],
    )(adj_t, uadj_t, alpha_t, uops_t)

    return (jnp.transpose(adj_out_t, (2, 0, 1)),
            jnp.transpose(ops_out_t, (2, 0, 1)))
